# Initial kernel scaffold; baseline (speedup 1.0000x reference)
import jax
import jax.numpy as jnp
from jax.experimental import pallas as pl


def kernel(x, edge_attr, params, edge_index, batch):
    raise NotImplementedError("write your pallas kernel here")


# speedup vs baseline: 20.7290x; 20.7290x over previous
import functools

import jax
import jax.numpy as jnp
from jax import lax
from jax.experimental import pallas as pl
from jax.experimental.pallas import tpu as pltpu
from jax.experimental.pallas import tpu_sc as plsc

N = 50000
E = 800000
F_NODE = 8
HID = 64
HEADS = 4
C = HID // HEADS
L = 4
OUT = 100
G = 8

NBLK = 256
NGRID = 196
N_PAD = NGRID * NBLK
HALF = N_PAD // 2
RPT = HALF // 16
ECHUNK = 128
EC_TC = 2048
NCH = 391
E_PAD = 16 * ECHUNK * NCH
F32 = jnp.float32


def _edge_pre_body(ea_ref, m3_ref, c3_ref, ae0, ae1, ae2, ae3, sum_ref):
    i = pl.program_id(0)
    ea = ea_ref[...]
    p = jnp.dot(ea, m3_ref[...], preferred_element_type=F32) + c3_ref[...]
    outs = (ae0, ae1, ae2, ae3)
    for l in range(L):
        outs[l][...] = p[:, l * 16:(l + 1) * 16]
    s = jnp.broadcast_to(jnp.sum(ea, axis=0, keepdims=True), (8, 8))

    @pl.when(i == 0)
    def _():
        sum_ref[...] = s

    @pl.when(i != 0)
    def _():
        sum_ref[...] = sum_ref[...] + s


def _edge_pre(ea_p, m3, c3):
    return pl.pallas_call(
        _edge_pre_body,
        grid=(E_PAD // EC_TC,),
        in_specs=[
            pl.BlockSpec((EC_TC, 8), lambda i: (i, 0)),
            pl.BlockSpec((8, HID), lambda i: (0, 0)),
            pl.BlockSpec((1, HID), lambda i: (0, 0)),
        ],
        out_specs=[pl.BlockSpec((EC_TC, 16), lambda i: (i, 0))] * L
        + [pl.BlockSpec((8, 8), lambda i: (0, 0))],
        out_shape=[jax.ShapeDtypeStruct((E_PAD, 16), F32)] * L
        + [jax.ShapeDtypeStruct((8, 8), F32)],
    )(ea_p, m3, c3)


def _attn_tail(hp, sm_ref, aloop_ref, rep_ref, hp_o, as_o, ad_o, inum_o, iden_o):
    s = jnp.dot(hp, sm_ref[...], preferred_element_type=F32)
    as16 = s[:, 0:16]
    ad16 = s[:, 16:32]
    alpha16 = as16 + ad16 + aloop_ref[...]
    wl16 = jnp.exp(jnp.maximum(alpha16, 0.2 * alpha16))
    wl64 = jnp.dot(wl16, rep_ref[...], preferred_element_type=F32)
    hp_o[...] = hp
    as_o[...] = as16
    ad_o[...] = ad16
    inum_o[...] = hp * wl64
    iden_o[...] = wl16


def _node0_body(x_ref, hpm_ref, hpb_ref, sm_ref, aloop_ref, rep_ref,
                hp_o, as_o, ad_o, inum_o, iden_o):
    hp = jnp.dot(x_ref[...], hpm_ref[...], preferred_element_type=F32) \
        + hpb_ref[...]
    _attn_tail(hp, sm_ref, aloop_ref, rep_ref, hp_o, as_o, ad_o, inum_o, iden_o)


def _node0(x_p, hpm, hpb, sm, aloop, rep):
    outs = [
        jax.ShapeDtypeStruct((N_PAD, HID), F32),
        jax.ShapeDtypeStruct((N_PAD, 16), F32),
        jax.ShapeDtypeStruct((N_PAD, 16), F32),
        jax.ShapeDtypeStruct((N_PAD, HID), F32),
        jax.ShapeDtypeStruct((N_PAD, 16), F32),
    ]
    return pl.pallas_call(
        _node0_body,
        grid=(NGRID,),
        in_specs=[
            pl.BlockSpec((NBLK, F_NODE), lambda i: (i, 0)),
            pl.BlockSpec((F_NODE, HID), lambda i: (0, 0)),
            pl.BlockSpec((1, HID), lambda i: (0, 0)),
            pl.BlockSpec((HID, 32), lambda i: (0, 0)),
            pl.BlockSpec((1, 16), lambda i: (0, 0)),
            pl.BlockSpec((16, HID), lambda i: (0, 0)),
        ],
        out_specs=[
            pl.BlockSpec((NBLK, HID), lambda i: (i, 0)),
            pl.BlockSpec((NBLK, 16), lambda i: (i, 0)),
            pl.BlockSpec((NBLK, 16), lambda i: (i, 0)),
            pl.BlockSpec((NBLK, HID), lambda i: (i, 0)),
            pl.BlockSpec((NBLK, 16), lambda i: (i, 0)),
        ],
        out_shape=outs,
    )(x_p, hpm, hpb, sm, aloop, rep)


def _post_h(num_ref, den_ref, rep_ref, bias_ref, g_ref, b_ref):
    den64 = jnp.dot(den_ref[...], rep_ref[...], preferred_element_type=F32)
    o = num_ref[...] / den64 + bias_ref[...]
    mu = jnp.mean(o, axis=1, keepdims=True)
    d0 = o - mu
    var = jnp.mean(d0 * d0, axis=1, keepdims=True)
    ln = d0 * lax.rsqrt(var + 1e-5) * g_ref[...] + b_ref[...]
    return jnp.maximum(ln, 0.0)


def _node_body(num_ref, den_ref, rep_ref, bias_ref, g_ref, b_ref, w_ref,
               sm_ref, aloop_ref, hp_o, as_o, ad_o, inum_o, iden_o):
    h = _post_h(num_ref, den_ref, rep_ref, bias_ref, g_ref, b_ref)
    hp = jnp.dot(h, w_ref[...], preferred_element_type=F32)
    _attn_tail(hp, sm_ref, aloop_ref, rep_ref, hp_o, as_o, ad_o, inum_o, iden_o)


def _node(num, den, rep, bias, g, b, w, sm, aloop):
    outs = [
        jax.ShapeDtypeStruct((N_PAD, HID), F32),
        jax.ShapeDtypeStruct((N_PAD, 16), F32),
        jax.ShapeDtypeStruct((N_PAD, 16), F32),
        jax.ShapeDtypeStruct((N_PAD, HID), F32),
        jax.ShapeDtypeStruct((N_PAD, 16), F32),
    ]
    return pl.pallas_call(
        _node_body,
        grid=(NGRID,),
        in_specs=[
            pl.BlockSpec((NBLK, HID), lambda i: (i, 0)),
            pl.BlockSpec((NBLK, 16), lambda i: (i, 0)),
            pl.BlockSpec((16, HID), lambda i: (0, 0)),
            pl.BlockSpec((1, HID), lambda i: (0, 0)),
            pl.BlockSpec((1, HID), lambda i: (0, 0)),
            pl.BlockSpec((1, HID), lambda i: (0, 0)),
            pl.BlockSpec((HID, HID), lambda i: (0, 0)),
            pl.BlockSpec((HID, 32), lambda i: (0, 0)),
            pl.BlockSpec((1, 16), lambda i: (0, 0)),
        ],
        out_specs=[
            pl.BlockSpec((NBLK, HID), lambda i: (i, 0)),
            pl.BlockSpec((NBLK, 16), lambda i: (i, 0)),
            pl.BlockSpec((NBLK, 16), lambda i: (i, 0)),
            pl.BlockSpec((NBLK, HID), lambda i: (i, 0)),
            pl.BlockSpec((NBLK, 16), lambda i: (i, 0)),
        ],
        out_shape=outs,
    )(num, den, rep, bias, g, b, w, sm, aloop)


def _pool_body(num_ref, den_ref, batch_ref, rep_ref, bias_ref, g_ref, b_ref,
               emb_ref, one_ref, pool_o):
    i = pl.program_id(0)
    h = _post_h(num_ref, den_ref, rep_ref, bias_ref, g_ref, b_ref)
    hext = jnp.dot(h, emb_ref[...], preferred_element_type=F32) + one_ref[...]
    brow = batch_ref[...]
    gio = lax.broadcasted_iota(jnp.int32, (128, NBLK), 0).astype(F32)
    oht = jnp.where(gio == jnp.broadcast_to(brow, (128, NBLK)), 1.0, 0.0)
    acc = jnp.dot(oht, hext, preferred_element_type=F32)

    @pl.when(i == 0)
    def _():
        pool_o[...] = acc

    @pl.when(i != 0)
    def _():
        pool_o[...] = pool_o[...] + acc


def _pool(num, den, batch2, rep, bias, g, b, emb, one):
    return pl.pallas_call(
        _pool_body,
        grid=(NGRID,),
        in_specs=[
            pl.BlockSpec((NBLK, HID), lambda i: (i, 0)),
            pl.BlockSpec((NBLK, 16), lambda i: (i, 0)),
            pl.BlockSpec((None, 1, NBLK), lambda i: (i, 0, 0)),
            pl.BlockSpec((16, HID), lambda i: (0, 0)),
            pl.BlockSpec((1, HID), lambda i: (0, 0)),
            pl.BlockSpec((1, HID), lambda i: (0, 0)),
            pl.BlockSpec((1, HID), lambda i: (0, 0)),
            pl.BlockSpec((HID, 128), lambda i: (0, 0)),
            pl.BlockSpec((1, 128), lambda i: (0, 0)),
        ],
        out_specs=pl.BlockSpec((128, 128), lambda i: (0, 0)),
        out_shape=jax.ShapeDtypeStruct((128, 128), F32),
    )(num, den, batch2, rep, bias, g, b, emb, one)


def _head_body(pool_ref, w1_ref, b1_ref, w2_ref, b2_ref, out_o):
    cnt = jnp.maximum(pool_ref[:, 64:65], 1.0)
    pooled = pool_ref[:, 0:64] / cnt
    hid = jnp.maximum(
        jnp.dot(pooled, w1_ref[...], preferred_element_type=F32) + b1_ref[...],
        0.0)
    out_o[...] = jnp.dot(hid, w2_ref[...], preferred_element_type=F32) \
        + b2_ref[...]


def _head(pool, w1, b1, w2, b2):
    return pl.pallas_call(
        _head_body,
        grid=(1,),
        in_specs=[
            pl.BlockSpec((128, 128), lambda i: (0, 0)),
            pl.BlockSpec((HID, HID), lambda i: (0, 0)),
            pl.BlockSpec((1, HID), lambda i: (0, 0)),
            pl.BlockSpec((HID, 256), lambda i: (0, 0)),
            pl.BlockSpec((1, 256), lambda i: (0, 0)),
        ],
        out_specs=pl.BlockSpec((128, 256), lambda i: (0, 0)),
        out_shape=jax.ShapeDtypeStruct((128, 256), F32),
    )(pool, w1, b1, w2, b2)


def _sc_num_body(srcp, dstp, ae, hpt, asxt, adxt, inum, num_out, w_out,
                 srcv, dstv, locv, aev, asv, adv, hv, msgv, wv, numacc):
    c = lax.axis_index("c")
    s = lax.axis_index("s")
    lo = c * HALF
    rbase = s * RPT

    pltpu.sync_copy(inum.at[pl.ds(lo + rbase, RPT)], numacc.at[pl.ds(rbase, RPT)])
    plsc.subcore_barrier()

    @pl.loop(0, NCH)
    def _chunk(j):
        base = (s * NCH + j) * ECHUNK
        pltpu.sync_copy(srcp.at[pl.ds(base, ECHUNK)], srcv)
        pltpu.sync_copy(dstp.at[pl.ds(base, ECHUNK)], dstv)
        pltpu.sync_copy(ae.at[pl.ds(base, ECHUNK)], aev)
        pltpu.sync_copy(hpt.at[srcv], hv)
        pltpu.sync_copy(asxt.at[srcv], asv)
        pltpu.sync_copy(adxt.at[dstv], adv)

        for i in range(ECHUNK // 16):
            d = dstv[pl.ds(i * 16, 16)]
            owned = (d >= lo) & (d < lo + HALF)
            locv[pl.ds(i * 16, 16)] = jnp.where(owned, d - lo, -1)

        @pl.loop(0, ECHUNK)
        def _edge(e):
            va = asv[e, :] + adv[e, :] + aev[e, :]
            w = jnp.exp(jnp.maximum(va, 0.2 * va))
            wv[e, :] = w
            for h in range(HEADS):
                msgv[e, pl.ds(h * 16, 16)] = w[h] * hv[e, pl.ds(h * 16, 16)]

        idx = plsc.Indices(locv, ignored_value=-1)
        pltpu.sync_copy(msgv, numacc.at[idx], add=True)

        @pl.when((j & 1) == c)
        def _():
            pltpu.sync_copy(wv, w_out.at[pl.ds(base, ECHUNK)])

    plsc.subcore_barrier()
    pltpu.sync_copy(numacc.at[pl.ds(rbase, RPT)], num_out.at[pl.ds(lo + rbase, RPT)])


def _sc_den_body(dstp, w_in, iden, den_out, dstv, locv, wv, denacc):
    c = lax.axis_index("c")
    s = lax.axis_index("s")
    lo = c * HALF
    rbase = s * RPT

    pltpu.sync_copy(iden.at[pl.ds(lo + rbase, RPT)], denacc.at[pl.ds(rbase, RPT)])
    plsc.subcore_barrier()

    @pl.loop(0, NCH)
    def _chunk(j):
        base = (s * NCH + j) * ECHUNK
        pltpu.sync_copy(dstp.at[pl.ds(base, ECHUNK)], dstv)
        pltpu.sync_copy(w_in.at[pl.ds(base, ECHUNK)], wv)
        for i in range(ECHUNK // 16):
            d = dstv[pl.ds(i * 16, 16)]
            owned = (d >= lo) & (d < lo + HALF)
            locv[pl.ds(i * 16, 16)] = jnp.where(owned, d - lo, -1)
        pltpu.sync_copy(wv, denacc.at[plsc.Indices(locv, ignored_value=-1)],
                        add=True)

    plsc.subcore_barrier()
    pltpu.sync_copy(denacc.at[pl.ds(rbase, RPT)], den_out.at[pl.ds(lo + rbase, RPT)])


def _sc_edge_pass(srcp, dstp, ae, hpt, asxt, adxt, inum, iden):
    mesh = plsc.VectorSubcoreMesh(core_axis_name="c", subcore_axis_name="s")
    num_fn = pl.kernel(
        _sc_num_body,
        out_type=(
            jax.ShapeDtypeStruct((N_PAD, HID), F32),
            jax.ShapeDtypeStruct((E_PAD, 16), F32),
        ),
        mesh=mesh,
        compiler_params=pltpu.CompilerParams(use_tc_tiling_on_sc=False),
        scratch_types=[
            pltpu.VMEM((ECHUNK,), jnp.int32),
            pltpu.VMEM((ECHUNK,), jnp.int32),
            pltpu.VMEM((ECHUNK,), jnp.int32),
            pltpu.VMEM((ECHUNK, 16), F32),
            pltpu.VMEM((ECHUNK, 16), F32),
            pltpu.VMEM((ECHUNK, 16), F32),
            pltpu.VMEM((ECHUNK, HID), F32),
            pltpu.VMEM((ECHUNK, HID), F32),
            pltpu.VMEM((ECHUNK, 16), F32),
            pltpu.VMEM_SHARED((HALF, HID), F32),
        ],
    )
    num, w = num_fn(srcp, dstp, ae, hpt, asxt, adxt, inum)
    den_fn = pl.kernel(
        _sc_den_body,
        out_type=jax.ShapeDtypeStruct((N_PAD, 16), F32),
        mesh=mesh,
        compiler_params=pltpu.CompilerParams(use_tc_tiling_on_sc=False),
        scratch_types=[
            pltpu.VMEM((ECHUNK,), jnp.int32),
            pltpu.VMEM((ECHUNK,), jnp.int32),
            pltpu.VMEM((ECHUNK, 16), F32),
            pltpu.VMEM_SHARED((HALF, 16), F32),
        ],
    )
    den = den_fn(dstp, w, iden)
    return num, den


def _prep_params(params):
    rep = (jnp.arange(16)[:, None] == (jnp.arange(HID) // 16)[None, :])
    rep = rep.astype(F32)
    eye4 = jnp.eye(HEADS, 16, dtype=F32)
    pre = {'rep': rep}
    m3_cols, c3_cols = [], []
    for l in range(L):
        we_r = params[f'we_{l}'].reshape(HID, HEADS, C)
        v = jnp.einsum('khc,hc->kh', we_r, params[f'att_edge_{l}'])
        m3 = params['edge_w'] @ v
        c3 = params['edge_b'] @ v
        m3 = jnp.pad(m3, ((0, 5), (0, 12)))
        m3_cols.append(m3.at[3, :].set(-30000.0))
        c3_cols.append(jnp.pad(c3, (0, 12)))
        pre[f'v_{l}'] = v
        as16 = jnp.einsum('hc,hg->chg', params[f'att_src_{l}'], eye4)
        as16 = as16.transpose(1, 0, 2).reshape(HID, 16)
        ad16 = jnp.einsum('hc,hg->chg', params[f'att_dst_{l}'], eye4)
        ad16 = ad16.transpose(1, 0, 2).reshape(HID, 16)
        pre[f'sm_{l}'] = jnp.concatenate([as16, ad16], axis=1)
    pre['m3'] = jnp.concatenate(m3_cols, axis=1)
    pre['c3'] = jnp.concatenate(c3_cols)[None, :]
    pre['hpm0'] = params['node_w'] @ params['w_0']
    pre['hpb0'] = (params['node_b'] @ params['w_0'])[None, :]
    pre['emb'] = jnp.pad(jnp.eye(HID, dtype=F32), ((0, 0), (0, 64)))
    pre['one'] = (jnp.arange(128) == HID).astype(F32)[None, :]
    w1 = jnp.concatenate([params['dos_w1'], params['trans_w1']], axis=1)
    b1 = jnp.concatenate([params['dos_b1'], params['trans_b1']])[None, :]
    w2 = jnp.zeros((HID, 256), F32)
    w2 = w2.at[0:32, 0:OUT].set(params['dos_w2'])
    w2 = w2.at[32:64, 128:128 + OUT].set(params['trans_w2'])
    b2 = jnp.zeros((1, 256), F32)
    b2 = b2.at[0, 0:OUT].set(params['dos_b2'])
    b2 = b2.at[0, 128:128 + OUT].set(params['trans_b2'])
    pre.update(w1=w1, b1=b1, w2=w2, b2=b2)
    return pre


def kernel(x, edge_attr, params, edge_index, batch):
    src = edge_index[0].astype(jnp.int32)
    dst = edge_index[1].astype(jnp.int32)
    srcp = jnp.pad(src, (0, E_PAD - E))
    dstp = jnp.pad(dst, (0, E_PAD - E))
    ea_p = jnp.pad(edge_attr.astype(F32), ((0, E_PAD - E), (0, 5)))
    ea_p = ea_p.at[E:, 3].set(1.0)
    x_p = jnp.pad(x.astype(F32), ((0, N_PAD - N), (0, 0)))
    batch2 = jnp.pad(batch.astype(F32), (0, N_PAD - N),
                     constant_values=-1.0).reshape(NGRID, 1, NBLK)

    pre = _prep_params(params)

    *aes, sum_ea = _edge_pre(ea_p, pre['m3'], pre['c3'])
    e_mean = (sum_ea[0] / E) @ jnp.pad(params['edge_w'], ((0, 5), (0, 0))) \
        + params['edge_b']
    aloops = []
    for l in range(L):
        al = e_mean @ pre[f'v_{l}']
        aloops.append(jnp.pad(al, (0, 12))[None, :])

    rep = pre['rep']
    hpt, asx, adx, inum, iden = _node0(
        x_p, pre['hpm0'], pre['hpb0'], pre['sm_0'], aloops[0], rep)

    num, den = None, None
    for l in range(L):
        num, den = _sc_edge_pass(srcp, dstp, aes[l], hpt, asx, adx, inum, iden)
        if l + 1 < L:
            bias = params[f'bias_{l}'][None, :]
            g = params[f'ln_g_{l}'][None, :]
            b = params[f'ln_b_{l}'][None, :]
            hpt, asx, adx, inum, iden = _node(
                num, den, rep, bias, g, b, params[f'w_{l + 1}'],
                pre[f'sm_{l + 1}'], aloops[l + 1])

    bias = params[f'bias_{L - 1}'][None, :]
    g = params[f'ln_g_{L - 1}'][None, :]
    b = params[f'ln_b_{L - 1}'][None, :]
    pool = _pool(num, den, batch2, rep, bias, g, b, pre['emb'], pre['one'])
    outm = _head(pool, pre['w1'], pre['b1'], pre['w2'], pre['b2'])
    dos = outm[0:G, 0:OUT]
    trans = outm[0:G, 128:128 + OUT]
    return (dos, trans)


# speedup vs baseline: 23.5759x; 1.1373x over previous
import functools

import jax
import jax.numpy as jnp
from jax import lax
from jax.experimental import pallas as pl
from jax.experimental.pallas import tpu as pltpu
from jax.experimental.pallas import tpu_sc as plsc

N = 50000
E = 800000
F_NODE = 8
HID = 64
HEADS = 4
C = HID // HEADS
L = 4
OUT = 100
G = 8

NBLK = 256
NGRID = 196
N_PAD = NGRID * NBLK
HALF = N_PAD // 2
RPT = HALF // 16
ECHUNK = 64
EC_TC = 2048
NCH = 782
E_PAD = 16 * ECHUNK * NCH
F32 = jnp.float32


def _edge_pre_body(ea_ref, m3_ref, c3_ref, ae0, ae1, ae2, ae3, sum_ref):
    i = pl.program_id(0)
    ea = ea_ref[...]
    p = jnp.dot(ea, m3_ref[...], preferred_element_type=F32) + c3_ref[...]
    outs = (ae0, ae1, ae2, ae3)
    for l in range(L):
        outs[l][...] = p[:, l * 16:(l + 1) * 16]
    s = jnp.broadcast_to(jnp.sum(ea, axis=0, keepdims=True), (8, 8))

    @pl.when(i == 0)
    def _():
        sum_ref[...] = s

    @pl.when(i != 0)
    def _():
        sum_ref[...] = sum_ref[...] + s


def _edge_pre(ea_p, m3, c3):
    return pl.pallas_call(
        _edge_pre_body,
        grid=(E_PAD // EC_TC,),
        in_specs=[
            pl.BlockSpec((EC_TC, 8), lambda i: (i, 0)),
            pl.BlockSpec((8, HID), lambda i: (0, 0)),
            pl.BlockSpec((1, HID), lambda i: (0, 0)),
        ],
        out_specs=[pl.BlockSpec((EC_TC, 16), lambda i: (i, 0))] * L
        + [pl.BlockSpec((8, 8), lambda i: (0, 0))],
        out_shape=[jax.ShapeDtypeStruct((E_PAD, 16), F32)] * L
        + [jax.ShapeDtypeStruct((8, 8), F32)],
    )(ea_p, m3, c3)


def _attn_tail(hp, sm_ref, aloop_ref, rep_ref, hp_o, as_o, ad_o, inum_o, iden_o):
    s = jnp.dot(hp, sm_ref[...], preferred_element_type=F32)
    as16 = s[:, 0:16]
    ad16 = s[:, 16:32]
    alpha16 = as16 + ad16 + aloop_ref[...]
    wl16 = jnp.exp(jnp.maximum(alpha16, 0.2 * alpha16))
    wl64 = jnp.dot(wl16, rep_ref[...], preferred_element_type=F32)
    hp_o[...] = hp
    as_o[...] = as16
    ad_o[...] = ad16
    inum_o[...] = hp * wl64
    iden_o[...] = wl16


def _node0_body(x_ref, hpm_ref, hpb_ref, sm_ref, aloop_ref, rep_ref,
                hp_o, as_o, ad_o, inum_o, iden_o):
    hp = jnp.dot(x_ref[...], hpm_ref[...], preferred_element_type=F32) \
        + hpb_ref[...]
    _attn_tail(hp, sm_ref, aloop_ref, rep_ref, hp_o, as_o, ad_o, inum_o, iden_o)


def _node0(x_p, hpm, hpb, sm, aloop, rep):
    outs = [
        jax.ShapeDtypeStruct((N_PAD, HID), F32),
        jax.ShapeDtypeStruct((N_PAD, 16), F32),
        jax.ShapeDtypeStruct((N_PAD, 16), F32),
        jax.ShapeDtypeStruct((N_PAD, HID), F32),
        jax.ShapeDtypeStruct((N_PAD, 16), F32),
    ]
    return pl.pallas_call(
        _node0_body,
        grid=(NGRID,),
        in_specs=[
            pl.BlockSpec((NBLK, F_NODE), lambda i: (i, 0)),
            pl.BlockSpec((F_NODE, HID), lambda i: (0, 0)),
            pl.BlockSpec((1, HID), lambda i: (0, 0)),
            pl.BlockSpec((HID, 32), lambda i: (0, 0)),
            pl.BlockSpec((1, 16), lambda i: (0, 0)),
            pl.BlockSpec((16, HID), lambda i: (0, 0)),
        ],
        out_specs=[
            pl.BlockSpec((NBLK, HID), lambda i: (i, 0)),
            pl.BlockSpec((NBLK, 16), lambda i: (i, 0)),
            pl.BlockSpec((NBLK, 16), lambda i: (i, 0)),
            pl.BlockSpec((NBLK, HID), lambda i: (i, 0)),
            pl.BlockSpec((NBLK, 16), lambda i: (i, 0)),
        ],
        out_shape=outs,
    )(x_p, hpm, hpb, sm, aloop, rep)


def _post_h(num_ref, den_ref, rep_ref, bias_ref, g_ref, b_ref):
    den64 = jnp.dot(den_ref[...], rep_ref[...], preferred_element_type=F32)
    o = num_ref[...] / den64 + bias_ref[...]
    mu = jnp.mean(o, axis=1, keepdims=True)
    d0 = o - mu
    var = jnp.mean(d0 * d0, axis=1, keepdims=True)
    ln = d0 * lax.rsqrt(var + 1e-5) * g_ref[...] + b_ref[...]
    return jnp.maximum(ln, 0.0)


def _node_body(num_ref, den_ref, rep_ref, bias_ref, g_ref, b_ref, w_ref,
               sm_ref, aloop_ref, hp_o, as_o, ad_o, inum_o, iden_o):
    h = _post_h(num_ref, den_ref, rep_ref, bias_ref, g_ref, b_ref)
    hp = jnp.dot(h, w_ref[...], preferred_element_type=F32)
    _attn_tail(hp, sm_ref, aloop_ref, rep_ref, hp_o, as_o, ad_o, inum_o, iden_o)


def _node(num, den, rep, bias, g, b, w, sm, aloop):
    outs = [
        jax.ShapeDtypeStruct((N_PAD, HID), F32),
        jax.ShapeDtypeStruct((N_PAD, 16), F32),
        jax.ShapeDtypeStruct((N_PAD, 16), F32),
        jax.ShapeDtypeStruct((N_PAD, HID), F32),
        jax.ShapeDtypeStruct((N_PAD, 16), F32),
    ]
    return pl.pallas_call(
        _node_body,
        grid=(NGRID,),
        in_specs=[
            pl.BlockSpec((NBLK, HID), lambda i: (i, 0)),
            pl.BlockSpec((NBLK, 16), lambda i: (i, 0)),
            pl.BlockSpec((16, HID), lambda i: (0, 0)),
            pl.BlockSpec((1, HID), lambda i: (0, 0)),
            pl.BlockSpec((1, HID), lambda i: (0, 0)),
            pl.BlockSpec((1, HID), lambda i: (0, 0)),
            pl.BlockSpec((HID, HID), lambda i: (0, 0)),
            pl.BlockSpec((HID, 32), lambda i: (0, 0)),
            pl.BlockSpec((1, 16), lambda i: (0, 0)),
        ],
        out_specs=[
            pl.BlockSpec((NBLK, HID), lambda i: (i, 0)),
            pl.BlockSpec((NBLK, 16), lambda i: (i, 0)),
            pl.BlockSpec((NBLK, 16), lambda i: (i, 0)),
            pl.BlockSpec((NBLK, HID), lambda i: (i, 0)),
            pl.BlockSpec((NBLK, 16), lambda i: (i, 0)),
        ],
        out_shape=outs,
    )(num, den, rep, bias, g, b, w, sm, aloop)


def _pool_body(num_ref, den_ref, batch_ref, rep_ref, bias_ref, g_ref, b_ref,
               emb_ref, one_ref, pool_o):
    i = pl.program_id(0)
    h = _post_h(num_ref, den_ref, rep_ref, bias_ref, g_ref, b_ref)
    hext = jnp.dot(h, emb_ref[...], preferred_element_type=F32) + one_ref[...]
    brow = batch_ref[...]
    gio = lax.broadcasted_iota(jnp.int32, (128, NBLK), 0).astype(F32)
    oht = jnp.where(gio == jnp.broadcast_to(brow, (128, NBLK)), 1.0, 0.0)
    acc = jnp.dot(oht, hext, preferred_element_type=F32)

    @pl.when(i == 0)
    def _():
        pool_o[...] = acc

    @pl.when(i != 0)
    def _():
        pool_o[...] = pool_o[...] + acc


def _pool(num, den, batch2, rep, bias, g, b, emb, one):
    return pl.pallas_call(
        _pool_body,
        grid=(NGRID,),
        in_specs=[
            pl.BlockSpec((NBLK, HID), lambda i: (i, 0)),
            pl.BlockSpec((NBLK, 16), lambda i: (i, 0)),
            pl.BlockSpec((None, 1, NBLK), lambda i: (i, 0, 0)),
            pl.BlockSpec((16, HID), lambda i: (0, 0)),
            pl.BlockSpec((1, HID), lambda i: (0, 0)),
            pl.BlockSpec((1, HID), lambda i: (0, 0)),
            pl.BlockSpec((1, HID), lambda i: (0, 0)),
            pl.BlockSpec((HID, 128), lambda i: (0, 0)),
            pl.BlockSpec((1, 128), lambda i: (0, 0)),
        ],
        out_specs=pl.BlockSpec((128, 128), lambda i: (0, 0)),
        out_shape=jax.ShapeDtypeStruct((128, 128), F32),
    )(num, den, batch2, rep, bias, g, b, emb, one)


def _head_body(pool_ref, w1_ref, b1_ref, w2_ref, b2_ref, out_o):
    cnt = jnp.maximum(pool_ref[:, 64:65], 1.0)
    pooled = pool_ref[:, 0:64] / cnt
    hid = jnp.maximum(
        jnp.dot(pooled, w1_ref[...], preferred_element_type=F32) + b1_ref[...],
        0.0)
    out_o[...] = jnp.dot(hid, w2_ref[...], preferred_element_type=F32) \
        + b2_ref[...]


def _head(pool, w1, b1, w2, b2):
    return pl.pallas_call(
        _head_body,
        grid=(1,),
        in_specs=[
            pl.BlockSpec((128, 128), lambda i: (0, 0)),
            pl.BlockSpec((HID, HID), lambda i: (0, 0)),
            pl.BlockSpec((1, HID), lambda i: (0, 0)),
            pl.BlockSpec((HID, 256), lambda i: (0, 0)),
            pl.BlockSpec((1, 256), lambda i: (0, 0)),
        ],
        out_specs=pl.BlockSpec((128, 256), lambda i: (0, 0)),
        out_shape=jax.ShapeDtypeStruct((128, 256), F32),
    )(pool, w1, b1, w2, b2)


def _sc_num_body(srcp, dstp, ae, hpt, asxt, adxt, inum, num_out, w_out,
                 srcv0, dstv0, aev0, hv0, asv0, adv0,
                 srcv1, dstv1, aev1, hv1, asv1, adv1,
                 locv, msgv, wv, numacc, gsem0, gsem1):
    c = lax.axis_index("c")
    s = lax.axis_index("s")
    lo = c * HALF
    rbase = s * RPT
    slots = ((srcv0, dstv0, aev0, hv0, asv0, adv0, gsem0),
             (srcv1, dstv1, aev1, hv1, asv1, adv1, gsem1))

    pltpu.sync_copy(inum.at[pl.ds(lo + rbase, RPT)], numacc.at[pl.ds(rbase, RPT)])
    plsc.subcore_barrier()

    def stage_in(k, p):
        srcv, dstv, aev, hv, asv, adv, gsem = slots[p]
        base = (s * NCH + k) * ECHUNK
        pltpu.sync_copy(srcp.at[pl.ds(base, ECHUNK)], srcv)
        pltpu.sync_copy(dstp.at[pl.ds(base, ECHUNK)], dstv)
        pltpu.sync_copy(ae.at[pl.ds(base, ECHUNK)], aev)
        pltpu.async_copy(hpt.at[srcv], hv, gsem)
        pltpu.async_copy(asxt.at[srcv], asv, gsem)
        pltpu.async_copy(adxt.at[dstv], adv, gsem)

    def consume(k, p):
        srcv, dstv, aev, hv, asv, adv, gsem = slots[p]
        pltpu.make_async_copy(hpt.at[srcv], hv, gsem).wait()
        pltpu.make_async_copy(asxt.at[srcv], asv, gsem).wait()
        pltpu.make_async_copy(adxt.at[dstv], adv, gsem).wait()

        for i in range(ECHUNK // 16):
            d = dstv[pl.ds(i * 16, 16)]
            owned = (d >= lo) & (d < lo + HALF)
            locv[pl.ds(i * 16, 16)] = jnp.where(owned, d - lo, -1)

        @pl.loop(0, ECHUNK)
        def _edge(e):
            va = asv[e, :] + adv[e, :] + aev[e, :]
            w = jnp.exp(jnp.maximum(va, 0.2 * va))
            wv[e, :] = w
            for h in range(HEADS):
                msgv[e, pl.ds(h * 16, 16)] = w[h] * hv[e, pl.ds(h * 16, 16)]

        idx = plsc.Indices(locv, ignored_value=-1)
        pltpu.sync_copy(msgv, numacc.at[idx], add=True)

        base = (s * NCH + k) * ECHUNK
        @pl.when((k & 1) == c)
        def _():
            pltpu.sync_copy(wv, w_out.at[pl.ds(base, ECHUNK)])

    stage_in(0, 0)

    @pl.loop(0, NCH, step=2)
    def _chunk(j):
        stage_in(j + 1, 1)
        consume(j, 0)

        @pl.when(j + 2 < NCH)
        def _():
            stage_in(j + 2, 0)
        consume(j + 1, 1)

    plsc.subcore_barrier()
    pltpu.sync_copy(numacc.at[pl.ds(rbase, RPT)], num_out.at[pl.ds(lo + rbase, RPT)])


def _sc_den_body(dstp, w_in, iden, den_out, dstv0, wv0, dstv1, wv1, locv,
                 denacc, lsem0, lsem1):
    c = lax.axis_index("c")
    s = lax.axis_index("s")
    lo = c * HALF
    rbase = s * RPT
    slots = ((dstv0, wv0, lsem0), (dstv1, wv1, lsem1))

    pltpu.sync_copy(iden.at[pl.ds(lo + rbase, RPT)], denacc.at[pl.ds(rbase, RPT)])
    plsc.subcore_barrier()

    def stage_in(k, p):
        dstv, wv, lsem = slots[p]
        base = (s * NCH + k) * ECHUNK
        pltpu.async_copy(dstp.at[pl.ds(base, ECHUNK)], dstv, lsem)
        pltpu.async_copy(w_in.at[pl.ds(base, ECHUNK)], wv, lsem)

    def consume(k, p):
        dstv, wv, lsem = slots[p]
        base = (s * NCH + k) * ECHUNK
        pltpu.make_async_copy(dstp.at[pl.ds(base, ECHUNK)], dstv, lsem).wait()
        pltpu.make_async_copy(w_in.at[pl.ds(base, ECHUNK)], wv, lsem).wait()
        for i in range(ECHUNK // 16):
            d = dstv[pl.ds(i * 16, 16)]
            owned = (d >= lo) & (d < lo + HALF)
            locv[pl.ds(i * 16, 16)] = jnp.where(owned, d - lo, -1)
        pltpu.sync_copy(wv, denacc.at[plsc.Indices(locv, ignored_value=-1)],
                        add=True)

    stage_in(0, 0)

    @pl.loop(0, NCH, step=2)
    def _chunk(j):
        stage_in(j + 1, 1)
        consume(j, 0)

        @pl.when(j + 2 < NCH)
        def _():
            stage_in(j + 2, 0)
        consume(j + 1, 1)

    plsc.subcore_barrier()
    pltpu.sync_copy(denacc.at[pl.ds(rbase, RPT)], den_out.at[pl.ds(lo + rbase, RPT)])


def _sc_edge_pass(srcp, dstp, ae, hpt, asxt, adxt, inum, iden):
    mesh = plsc.VectorSubcoreMesh(core_axis_name="c", subcore_axis_name="s")
    num_fn = pl.kernel(
        _sc_num_body,
        out_type=(
            jax.ShapeDtypeStruct((N_PAD, HID), F32),
            jax.ShapeDtypeStruct((E_PAD, 16), F32),
        ),
        mesh=mesh,
        compiler_params=pltpu.CompilerParams(use_tc_tiling_on_sc=False),
        scratch_types=[
            pltpu.VMEM((ECHUNK,), jnp.int32),
            pltpu.VMEM((ECHUNK,), jnp.int32),
            pltpu.VMEM((ECHUNK, 16), F32),
            pltpu.VMEM((ECHUNK, HID), F32),
            pltpu.VMEM((ECHUNK, 16), F32),
            pltpu.VMEM((ECHUNK, 16), F32),
            pltpu.VMEM((ECHUNK,), jnp.int32),
            pltpu.VMEM((ECHUNK,), jnp.int32),
            pltpu.VMEM((ECHUNK, 16), F32),
            pltpu.VMEM((ECHUNK, HID), F32),
            pltpu.VMEM((ECHUNK, 16), F32),
            pltpu.VMEM((ECHUNK, 16), F32),
            pltpu.VMEM((ECHUNK,), jnp.int32),
            pltpu.VMEM((ECHUNK, HID), F32),
            pltpu.VMEM((ECHUNK, 16), F32),
            pltpu.VMEM_SHARED((HALF, HID), F32),
            pltpu.SemaphoreType.DMA,
            pltpu.SemaphoreType.DMA,
        ],
    )
    num, w = num_fn(srcp, dstp, ae, hpt, asxt, adxt, inum)
    den_fn = pl.kernel(
        _sc_den_body,
        out_type=jax.ShapeDtypeStruct((N_PAD, 16), F32),
        mesh=mesh,
        compiler_params=pltpu.CompilerParams(use_tc_tiling_on_sc=False),
        scratch_types=[
            pltpu.VMEM((ECHUNK,), jnp.int32),
            pltpu.VMEM((ECHUNK, 16), F32),
            pltpu.VMEM((ECHUNK,), jnp.int32),
            pltpu.VMEM((ECHUNK, 16), F32),
            pltpu.VMEM((ECHUNK,), jnp.int32),
            pltpu.VMEM_SHARED((HALF, 16), F32),
            pltpu.SemaphoreType.DMA,
            pltpu.SemaphoreType.DMA,
        ],
    )
    den = den_fn(dstp, w, iden)
    return num, den


def _prep_params(params):
    rep = (jnp.arange(16)[:, None] == (jnp.arange(HID) // 16)[None, :])
    rep = rep.astype(F32)
    eye4 = jnp.eye(HEADS, 16, dtype=F32)
    pre = {'rep': rep}
    m3_cols, c3_cols = [], []
    for l in range(L):
        we_r = params[f'we_{l}'].reshape(HID, HEADS, C)
        v = jnp.einsum('khc,hc->kh', we_r, params[f'att_edge_{l}'])
        m3 = params['edge_w'] @ v
        c3 = params['edge_b'] @ v
        m3 = jnp.pad(m3, ((0, 5), (0, 12)))
        m3_cols.append(m3.at[3, :].set(-30000.0))
        c3_cols.append(jnp.pad(c3, (0, 12)))
        pre[f'v_{l}'] = v
        as16 = jnp.einsum('hc,hg->chg', params[f'att_src_{l}'], eye4)
        as16 = as16.transpose(1, 0, 2).reshape(HID, 16)
        ad16 = jnp.einsum('hc,hg->chg', params[f'att_dst_{l}'], eye4)
        ad16 = ad16.transpose(1, 0, 2).reshape(HID, 16)
        pre[f'sm_{l}'] = jnp.concatenate([as16, ad16], axis=1)
    pre['m3'] = jnp.concatenate(m3_cols, axis=1)
    pre['c3'] = jnp.concatenate(c3_cols)[None, :]
    pre['hpm0'] = params['node_w'] @ params['w_0']
    pre['hpb0'] = (params['node_b'] @ params['w_0'])[None, :]
    pre['emb'] = jnp.pad(jnp.eye(HID, dtype=F32), ((0, 0), (0, 64)))
    pre['one'] = (jnp.arange(128) == HID).astype(F32)[None, :]
    w1 = jnp.concatenate([params['dos_w1'], params['trans_w1']], axis=1)
    b1 = jnp.concatenate([params['dos_b1'], params['trans_b1']])[None, :]
    w2 = jnp.zeros((HID, 256), F32)
    w2 = w2.at[0:32, 0:OUT].set(params['dos_w2'])
    w2 = w2.at[32:64, 128:128 + OUT].set(params['trans_w2'])
    b2 = jnp.zeros((1, 256), F32)
    b2 = b2.at[0, 0:OUT].set(params['dos_b2'])
    b2 = b2.at[0, 128:128 + OUT].set(params['trans_b2'])
    pre.update(w1=w1, b1=b1, w2=w2, b2=b2)
    return pre


def kernel(x, edge_attr, params, edge_index, batch):
    src = edge_index[0].astype(jnp.int32)
    dst = edge_index[1].astype(jnp.int32)
    srcp = jnp.pad(src, (0, E_PAD - E))
    dstp = jnp.pad(dst, (0, E_PAD - E))
    ea_p = jnp.pad(edge_attr.astype(F32), ((0, E_PAD - E), (0, 5)))
    ea_p = ea_p.at[E:, 3].set(1.0)
    x_p = jnp.pad(x.astype(F32), ((0, N_PAD - N), (0, 0)))
    batch2 = jnp.pad(batch.astype(F32), (0, N_PAD - N),
                     constant_values=-1.0).reshape(NGRID, 1, NBLK)

    pre = _prep_params(params)

    *aes, sum_ea = _edge_pre(ea_p, pre['m3'], pre['c3'])
    e_mean = (sum_ea[0] / E) @ jnp.pad(params['edge_w'], ((0, 5), (0, 0))) \
        + params['edge_b']
    aloops = []
    for l in range(L):
        al = e_mean @ pre[f'v_{l}']
        aloops.append(jnp.pad(al, (0, 12))[None, :])

    rep = pre['rep']
    hpt, asx, adx, inum, iden = _node0(
        x_p, pre['hpm0'], pre['hpb0'], pre['sm_0'], aloops[0], rep)

    num, den = None, None
    for l in range(L):
        num, den = _sc_edge_pass(srcp, dstp, aes[l], hpt, asx, adx, inum, iden)
        if l + 1 < L:
            bias = params[f'bias_{l}'][None, :]
            g = params[f'ln_g_{l}'][None, :]
            b = params[f'ln_b_{l}'][None, :]
            hpt, asx, adx, inum, iden = _node(
                num, den, rep, bias, g, b, params[f'w_{l + 1}'],
                pre[f'sm_{l + 1}'], aloops[l + 1])

    bias = params[f'bias_{L - 1}'][None, :]
    g = params[f'ln_g_{L - 1}'][None, :]
    b = params[f'ln_b_{L - 1}'][None, :]
    pool = _pool(num, den, batch2, rep, bias, g, b, pre['emb'], pre['one'])
    outm = _head(pool, pre['w1'], pre['b1'], pre['w2'], pre['b2'])
    dos = outm[0:G, 0:OUT]
    trans = outm[0:G, 128:128 + OUT]
    return (dos, trans)
